# baseline (speedup 1.0000x reference)
import functools

import jax
import jax.numpy as jnp
from jax import lax
from jax.experimental import pallas as pl
from jax.experimental.pallas import tpu as pltpu
from jax.experimental.pallas import tpu_sc as plsc

N = 10000
E = 320000
DIN = 128
DE = 16
D = 128
P_NODE = 0.1
P_EDGE = 0.1

NACC = 10112
CHUNK = 128
NW = 32
NCHW = 80
NCHP = NW * NCHW
EPAD = NCHP * CHUNK
RPT = NACC // 16


def _embed_body(a_ref, w_ref, m_ref, o_ref):
    o_ref[...] = jnp.dot(a_ref[...], w_ref[...],
                         preferred_element_type=jnp.float32) * m_ref[...]


def _embed(a, w, m, bm):
    r, k = a.shape
    return pl.pallas_call(
        _embed_body,
        grid=(r // bm,),
        in_specs=[
            pl.BlockSpec((bm, k), lambda i: (i, 0)),
            pl.BlockSpec((k, D), lambda i: (0, 0)),
            pl.BlockSpec((bm, 1), lambda i: (i, 0)),
        ],
        out_specs=pl.BlockSpec((bm, D), lambda i: (i, 0)),
        out_shape=jax.ShapeDtypeStruct((r, D), jnp.float32),
    )(a, w, m)


def _eupd_body(e_ref, w_ref, o_ref):
    e = e_ref[...]
    o_ref[...] = jnp.maximum(
        jnp.dot(e, w_ref[...], preferred_element_type=jnp.float32), 0.0) + e


def _e_update(e, w, bm=4096):
    return pl.pallas_call(
        _eupd_body,
        grid=(EPAD // bm,),
        in_specs=[
            pl.BlockSpec((bm, D), lambda i: (i, 0)),
            pl.BlockSpec((D, D), lambda i: (0, 0)),
        ],
        out_specs=pl.BlockSpec((bm, D), lambda i: (i, 0)),
        out_shape=jax.ShapeDtypeStruct((EPAD, D), jnp.float32),
    )(e, w)


def _hupd_body(p_ref, w_ref, r_ref, o_ref):
    agg = p_ref[0] + p_ref[1]
    o_ref[...] = jnp.maximum(
        jnp.dot(agg, w_ref[...], preferred_element_type=jnp.float32),
        0.0) + r_ref[...]


def _h_update(parts, w, res, bm=2000):
    return pl.pallas_call(
        _hupd_body,
        grid=(N // bm,),
        in_specs=[
            pl.BlockSpec((2, bm, D), lambda i: (0, i, 0)),
            pl.BlockSpec((D, D), lambda i: (0, 0)),
            pl.BlockSpec((bm, D), lambda i: (i, 0)),
        ],
        out_specs=pl.BlockSpec((bm, D), lambda i: (i, 0)),
        out_shape=jax.ShapeDtypeStruct((N, D), jnp.float32),
    )(parts, w, res)


def _add2_body(a_ref, b_ref, o_ref, *, scale):
    o_ref[...] = a_ref[...] + b_ref[...] * scale


def _add2(a, b, scale, bm=2000):
    return pl.pallas_call(
        functools.partial(_add2_body, scale=scale),
        grid=(N // bm,),
        in_specs=[
            pl.BlockSpec((bm, D), lambda i: (i, 0)),
            pl.BlockSpec((bm, D), lambda i: (i, 0)),
        ],
        out_specs=pl.BlockSpec((bm, D), lambda i: (i, 0)),
        out_shape=jax.ShapeDtypeStruct((N, D), jnp.float32),
    )(a, b)


def _sc_agg_body(h_hbm, e_hbm, src_hbm, dst_hbm, out_hbm,
                 e_buf, h_buf, sidx, didx, acc,
                 sem_e, sem_h, sem_i, sem_s):
    c = lax.axis_index("c")
    s = lax.axis_index("s")
    w = s * 2 + c
    base = w * NCHW
    zcopies = [CHUNK] * (RPT // CHUNK) + ([RPT % CHUNK] if RPT % CHUNK else [])

    def zrow(r, _):
        for k in range(D // 16):
            e_buf[0, r, pl.ds(k * 16, 16)] = jnp.zeros((16,), jnp.float32)
        return 0
    lax.fori_loop(0, CHUNK, zrow, 0)
    off = 0
    for zc in zcopies:
        pltpu.sync_copy(e_buf.at[0].at[pl.ds(0, zc)],
                        acc.at[pl.ds(s * RPT + off, zc)])
        off += zc
    plsc.subcore_barrier()

    def issue_e(t):
        pltpu.async_copy(
            e_hbm.at[pl.ds(jnp.minimum(base + t, NCHP - 1) * CHUNK, CHUNK)],
            e_buf.at[t % 2], sem_e.at[t % 2])

    def wait_e(t):
        pltpu.make_async_copy(e_hbm.at[pl.ds(0, CHUNK)], e_buf.at[t % 2],
                              sem_e.at[t % 2]).wait()

    def issue_idx(t):
        cid = jnp.minimum(base + t, NCHP - 1)
        j = t % 3
        pltpu.async_copy(src_hbm.at[cid], sidx.at[j], sem_i.at[j])
        pltpu.async_copy(dst_hbm.at[cid], didx.at[j], sem_i.at[j])

    def wait_idx(t):
        j = t % 3
        pltpu.make_async_copy(src_hbm.at[0], sidx.at[j], sem_i.at[j]).wait()
        pltpu.make_async_copy(dst_hbm.at[0], didx.at[j], sem_i.at[j]).wait()

    def issue_gather(t):
        wait_idx(t)
        pltpu.async_copy(h_hbm.at[sidx.at[t % 3]], h_buf, sem_h)

    def wait_gather(t):
        pltpu.make_async_copy(h_hbm.at[sidx.at[0]], h_buf, sem_h).wait()

    def wait_scatter(t):
        pltpu.make_async_copy(e_buf.at[t % 2], acc.at[didx.at[t % 3]],
                              sem_s.at[t % 2]).wait()

    issue_e(0)
    issue_idx(0)
    issue_idx(1)
    issue_gather(0)

    @pl.loop(0, NCHW)
    def phase(t):
        b = t % 2
        wait_gather(t)
        wait_e(t)

        def compute(slot):
            def mrow(r, _):
                for k in range(D // 16):
                    sl = pl.ds(k * 16, 16)
                    e_buf[slot, r, sl] = jnp.maximum(
                        e_buf[slot, r, sl] + h_buf[r, sl], 0.0)
                return 0
            lax.fori_loop(0, CHUNK, mrow, 0)

        pl.when(b == 0)(lambda: compute(0))
        pl.when(b == 1)(lambda: compute(1))

        issue_gather(t + 1)
        pltpu.async_copy(e_buf.at[b], acc.at[didx.at[t % 3]],
                         sem_s.at[b], add=True)

        @pl.when(t >= 1)
        def _():
            wait_scatter(t - 1)
        issue_e(t + 1)
        issue_idx(t + 2)

    wait_e(NCHW)
    wait_gather(NCHW)
    wait_idx(NCHW + 1)
    wait_scatter(NCHW - 1)
    plsc.subcore_barrier()

    off = 0
    for zc in zcopies:
        rows = pl.ds(s * RPT + off, zc)
        pltpu.sync_copy(acc.at[rows], out_hbm.at[c].at[rows])
        off += zc


def _sc_agg(h, e, src2d, dst2d):
    mesh = plsc.VectorSubcoreMesh(core_axis_name="c", subcore_axis_name="s")
    f = pl.kernel(
        _sc_agg_body,
        mesh=mesh,
        out_type=jax.ShapeDtypeStruct((2, NACC, D), jnp.float32),
        scratch_types=[
            pltpu.VMEM((2, CHUNK, D), jnp.float32),
            pltpu.VMEM((CHUNK, D), jnp.float32),
            pltpu.VMEM((3, CHUNK), jnp.int32),
            pltpu.VMEM((3, CHUNK), jnp.int32),
            pltpu.VMEM_SHARED((NACC, D), jnp.float32),
            pltpu.SemaphoreType.DMA((2,)),
            pltpu.SemaphoreType.DMA,
            pltpu.SemaphoreType.DMA((3,)),
            pltpu.SemaphoreType.DMA((2,)),
        ],
    )
    return f(h, e, src2d, dst2d)


def kernel(x, edge_index, edge_attr, W_atom, W_bond, Wn, We, node_rand, edge_rand):
    pad = EPAD - E
    iot = jnp.arange(pad, dtype=jnp.int32)
    src2d = jnp.concatenate(
        [edge_index[0], iot % N]).reshape(NCHP, CHUNK)
    dst2d = jnp.concatenate(
        [edge_index[1], N + (iot % (NACC - N))]).reshape(NCHP, CHUNK)
    node_mask = (node_rand > P_NODE).astype(jnp.float32)[:, None]
    edge_mask = (edge_rand > P_EDGE).astype(jnp.float32)[:, None]
    ea_pad = jnp.concatenate(
        [edge_attr, jnp.zeros((pad, DE), jnp.float32)])
    em_pad = jnp.concatenate([edge_mask, jnp.zeros((pad, 1), jnp.float32)])

    h0 = _embed(x, W_atom, node_mask, bm=2000)
    e0 = _embed(ea_pad, W_bond, em_pad, bm=4096)

    def mp(h, e, i):
        e2 = _e_update(e, We[i])
        parts = _sc_agg(h, e, src2d, dst2d)
        return _h_update(parts, Wn[i], h), e2

    h1, e1 = mp(h0, e0, 0)
    h2, e2 = mp(h1, e1, 1)
    g = _add2(h2, h1, 1.0)
    u00h, u00e = mp(g, e2, 3)
    xs0 = _add2(h1, u00h, 1.0)
    h3, e3 = mp(h2, e2, 2)
    g = _add2(h3, h2, 1.0)
    u11h, u11e = mp(g, e3, 5)
    g = _add2(u11h, xs0, 0.5)
    u10h, u10e = mp(g, u11e, 4)

    return (jnp.stack([u00h, u10h]), jnp.stack([u00e[:E], u10e[:E]]),
            node_mask.reshape(-1), edge_mask.reshape(-1))


# speedup vs baseline: 1.1329x; 1.1329x over previous
import functools

import jax
import jax.numpy as jnp
from jax import lax
from jax.experimental import pallas as pl
from jax.experimental.pallas import tpu as pltpu
from jax.experimental.pallas import tpu_sc as plsc

N = 10000
E = 320000
DIN = 128
DE = 16
D = 128
P_NODE = 0.1
P_EDGE = 0.1

NACC = 10112
CHUNK = 128
NW = 32
NCHW = 80
NCHUNKS = E // CHUNK
NCHP = NW * NCHW
RPT = NACC // 16


def _embed_body(a_ref, w_ref, m_ref, o_ref):
    o_ref[...] = jnp.dot(a_ref[...], w_ref[...],
                         preferred_element_type=jnp.float32) * m_ref[...]


def _embed(a, w, m, bm):
    r, k = a.shape
    return pl.pallas_call(
        _embed_body,
        grid=(r // bm,),
        in_specs=[
            pl.BlockSpec((bm, k), lambda i: (i, 0)),
            pl.BlockSpec((k, D), lambda i: (0, 0)),
            pl.BlockSpec((bm, 1), lambda i: (i, 0)),
        ],
        out_specs=pl.BlockSpec((bm, D), lambda i: (i, 0)),
        out_shape=jax.ShapeDtypeStruct((r, D), jnp.float32),
    )(a, w, m)


def _eupd_body(e_ref, w_ref, o_ref):
    e = e_ref[...]
    o_ref[...] = jnp.maximum(
        jnp.dot(e, w_ref[...], preferred_element_type=jnp.float32), 0.0) + e


def _e_update(e, w, bm=4000):
    return pl.pallas_call(
        _eupd_body,
        grid=(E // bm,),
        in_specs=[
            pl.BlockSpec((bm, D), lambda i: (i, 0)),
            pl.BlockSpec((D, D), lambda i: (0, 0)),
        ],
        out_specs=pl.BlockSpec((bm, D), lambda i: (i, 0)),
        out_shape=jax.ShapeDtypeStruct((E, D), jnp.float32),
    )(e, w)


def _hupd_body(p_ref, w_ref, r_ref, o_ref):
    agg = p_ref[0] + p_ref[1]
    o_ref[...] = jnp.maximum(
        jnp.dot(agg, w_ref[...], preferred_element_type=jnp.float32),
        0.0) + r_ref[...]


def _h_update(parts, w, res, bm=2000):
    return pl.pallas_call(
        _hupd_body,
        grid=(N // bm,),
        in_specs=[
            pl.BlockSpec((2, bm, D), lambda i: (0, i, 0)),
            pl.BlockSpec((D, D), lambda i: (0, 0)),
            pl.BlockSpec((bm, D), lambda i: (i, 0)),
        ],
        out_specs=pl.BlockSpec((bm, D), lambda i: (i, 0)),
        out_shape=jax.ShapeDtypeStruct((N, D), jnp.float32),
    )(parts, w, res)


def _add2_body(a_ref, b_ref, o_ref, *, scale):
    o_ref[...] = a_ref[...] + b_ref[...] * scale


def _add2(a, b, scale, bm=2000):
    return pl.pallas_call(
        functools.partial(_add2_body, scale=scale),
        grid=(N // bm,),
        in_specs=[
            pl.BlockSpec((bm, D), lambda i: (i, 0)),
            pl.BlockSpec((bm, D), lambda i: (i, 0)),
        ],
        out_specs=pl.BlockSpec((bm, D), lambda i: (i, 0)),
        out_shape=jax.ShapeDtypeStruct((N, D), jnp.float32),
    )(a, b)


def _sc_agg_body(h_hbm, e_hbm, src_hbm, dst_hbm, out_hbm,
                 e_buf, h_buf, sidx, didx, acc,
                 sem_e, sem_h, sem_i, sem_s):
    c = lax.axis_index("c")
    s = lax.axis_index("s")
    w = s * 2 + c
    base = w * NCHW
    zcopies = [CHUNK] * (RPT // CHUNK) + ([RPT % CHUNK] if RPT % CHUNK else [])

    def zrow(r, _):
        for k in range(D // 16):
            e_buf[0, r, pl.ds(k * 16, 16)] = jnp.zeros((16,), jnp.float32)
        return 0
    lax.fori_loop(0, CHUNK, zrow, 0)
    off = 0
    for zc in zcopies:
        pltpu.sync_copy(e_buf.at[0].at[pl.ds(0, zc)],
                        acc.at[pl.ds(s * RPT + off, zc)])
        off += zc
    plsc.subcore_barrier()

    def issue_e(t):
        pltpu.async_copy(
            e_hbm.at[pl.ds(jnp.minimum(base + t, NCHUNKS - 1) * CHUNK, CHUNK)],
            e_buf.at[t % 2], sem_e.at[t % 2])

    def wait_e(t):
        pltpu.make_async_copy(e_hbm.at[pl.ds(0, CHUNK)], e_buf.at[t % 2],
                              sem_e.at[t % 2]).wait()

    def issue_idx(t):
        cid = jnp.minimum(base + t, NCHP - 1)
        j = t % 3
        pltpu.async_copy(src_hbm.at[cid], sidx.at[j], sem_i.at[j])
        pltpu.async_copy(dst_hbm.at[cid], didx.at[j], sem_i.at[j])

    def wait_idx(t):
        j = t % 3
        pltpu.make_async_copy(src_hbm.at[0], sidx.at[j], sem_i.at[j]).wait()
        pltpu.make_async_copy(dst_hbm.at[0], didx.at[j], sem_i.at[j]).wait()

    def issue_gather(t):
        wait_idx(t)
        pltpu.async_copy(h_hbm.at[sidx.at[t % 3]], h_buf, sem_h)

    def wait_gather(t):
        pltpu.make_async_copy(h_hbm.at[sidx.at[0]], h_buf, sem_h).wait()

    def wait_scatter(t):
        pltpu.make_async_copy(e_buf.at[t % 2], acc.at[didx.at[t % 3]],
                              sem_s.at[t % 2]).wait()

    issue_e(0)
    issue_idx(0)
    issue_idx(1)
    issue_gather(0)

    @pl.loop(0, NCHW)
    def phase(t):
        b = t % 2
        wait_gather(t)
        wait_e(t)

        def compute(slot):
            def mrow(r, _):
                for k in range(D // 16):
                    sl = pl.ds(k * 16, 16)
                    e_buf[slot, r, sl] = jnp.maximum(
                        e_buf[slot, r, sl] + h_buf[r, sl], 0.0)
                return 0
            lax.fori_loop(0, CHUNK, mrow, 0)

        pl.when(b == 0)(lambda: compute(0))
        pl.when(b == 1)(lambda: compute(1))

        issue_gather(t + 1)
        pltpu.async_copy(e_buf.at[b], acc.at[didx.at[t % 3]],
                         sem_s.at[b], add=True)

        @pl.when(t >= 1)
        def _():
            wait_scatter(t - 1)
        issue_e(t + 1)
        issue_idx(t + 2)

    wait_e(NCHW)
    wait_gather(NCHW)
    wait_idx(NCHW + 1)
    wait_scatter(NCHW - 1)
    plsc.subcore_barrier()

    off = 0
    for zc in zcopies:
        rows = pl.ds(s * RPT + off, zc)
        pltpu.sync_copy(acc.at[rows], out_hbm.at[c].at[rows])
        off += zc


def _sc_agg(h, e, src2d, dst2d):
    mesh = plsc.VectorSubcoreMesh(core_axis_name="c", subcore_axis_name="s")
    f = pl.kernel(
        _sc_agg_body,
        mesh=mesh,
        out_type=jax.ShapeDtypeStruct((2, NACC, D), jnp.float32),
        scratch_types=[
            pltpu.VMEM((2, CHUNK, D), jnp.float32),
            pltpu.VMEM((CHUNK, D), jnp.float32),
            pltpu.VMEM((3, CHUNK), jnp.int32),
            pltpu.VMEM((3, CHUNK), jnp.int32),
            pltpu.VMEM_SHARED((NACC, D), jnp.float32),
            pltpu.SemaphoreType.DMA((2,)),
            pltpu.SemaphoreType.DMA,
            pltpu.SemaphoreType.DMA((3,)),
            pltpu.SemaphoreType.DMA((2,)),
        ],
    )
    return f(h, e, src2d, dst2d)


def kernel(x, edge_index, edge_attr, W_atom, W_bond, Wn, We, node_rand, edge_rand):
    pad = NCHP * CHUNK - E
    iot = jnp.arange(pad, dtype=jnp.int32)
    src2d = jnp.concatenate(
        [edge_index[0], iot % N]).reshape(NCHP, CHUNK)
    dst2d = jnp.concatenate(
        [edge_index[1], N + (iot % (NACC - N))]).reshape(NCHP, CHUNK)
    node_mask = (node_rand > P_NODE).astype(jnp.float32)[:, None]
    edge_mask = (edge_rand > P_EDGE).astype(jnp.float32)[:, None]

    h0 = _embed(x, W_atom, node_mask, bm=2000)
    e0 = _embed(edge_attr, W_bond, edge_mask, bm=4000)

    def mp(h, e, i):
        parts = _sc_agg(h, e, src2d, dst2d)
        return _h_update(parts, Wn[i], h), _e_update(e, We[i])

    h1, e1 = mp(h0, e0, 0)
    h2, e2 = mp(h1, e1, 1)
    g = _add2(h2, h1, 1.0)
    u00h, u00e = mp(g, e2, 3)
    xs0 = _add2(h1, u00h, 1.0)
    h3, e3 = mp(h2, e2, 2)
    g = _add2(h3, h2, 1.0)
    u11h, u11e = mp(g, e3, 5)
    g = _add2(u11h, xs0, 0.5)
    u10h, u10e = mp(g, u11e, 4)

    return (jnp.stack([u00h, u10h]), jnp.stack([u00e, u10e]),
            node_mask.reshape(-1), edge_mask.reshape(-1))


# speedup vs baseline: 1.1351x; 1.0020x over previous
import functools

import jax
import jax.numpy as jnp
from jax import lax
from jax.experimental import pallas as pl
from jax.experimental.pallas import tpu as pltpu
from jax.experimental.pallas import tpu_sc as plsc

N = 10000
E = 320000
DIN = 128
DE = 16
D = 128
P_NODE = 0.1
P_EDGE = 0.1

NACC = 10112
CHUNK = 128
NW = 32
NCHW = 80
NCHUNKS = E // CHUNK
NCHP = NW * NCHW
RPT = NACC // 16


def _embed_body(a_ref, w_ref, m_ref, o_ref):
    o_ref[...] = jnp.dot(a_ref[...], w_ref[...],
                         preferred_element_type=jnp.float32) * m_ref[...]


def _embed(a, w, m, bm):
    r, k = a.shape
    return pl.pallas_call(
        _embed_body,
        grid=(r // bm,),
        in_specs=[
            pl.BlockSpec((bm, k), lambda i: (i, 0)),
            pl.BlockSpec((k, D), lambda i: (0, 0)),
            pl.BlockSpec((bm, 1), lambda i: (i, 0)),
        ],
        out_specs=pl.BlockSpec((bm, D), lambda i: (i, 0)),
        out_shape=jax.ShapeDtypeStruct((r, D), jnp.float32),
    )(a, w, m)


def _eupd_body(e_ref, w_ref, o_ref):
    e = e_ref[...]
    o_ref[...] = jnp.maximum(
        jnp.dot(e, w_ref[...], preferred_element_type=jnp.float32), 0.0) + e


def _e_update(e, w, bm=4000):
    return pl.pallas_call(
        _eupd_body,
        grid=(E // bm,),
        in_specs=[
            pl.BlockSpec((bm, D), lambda i: (i, 0)),
            pl.BlockSpec((D, D), lambda i: (0, 0)),
        ],
        out_specs=pl.BlockSpec((bm, D), lambda i: (i, 0)),
        out_shape=jax.ShapeDtypeStruct((E, D), jnp.float32),
    )(e, w)


def _hupd_body(p_ref, w_ref, r_ref, o_ref):
    agg = p_ref[0] + p_ref[1]
    o_ref[...] = jnp.maximum(
        jnp.dot(agg, w_ref[...], preferred_element_type=jnp.float32),
        0.0) + r_ref[...]


def _h_update(parts, w, res, bm=2000):
    return pl.pallas_call(
        _hupd_body,
        grid=(N // bm,),
        in_specs=[
            pl.BlockSpec((2, bm, D), lambda i: (0, i, 0)),
            pl.BlockSpec((D, D), lambda i: (0, 0)),
            pl.BlockSpec((bm, D), lambda i: (i, 0)),
        ],
        out_specs=pl.BlockSpec((bm, D), lambda i: (i, 0)),
        out_shape=jax.ShapeDtypeStruct((N, D), jnp.float32),
    )(parts, w, res)


def _add2_body(a_ref, b_ref, o_ref, *, scale):
    o_ref[...] = a_ref[...] + b_ref[...] * scale


def _add2(a, b, scale, bm=2000):
    return pl.pallas_call(
        functools.partial(_add2_body, scale=scale),
        grid=(N // bm,),
        in_specs=[
            pl.BlockSpec((bm, D), lambda i: (i, 0)),
            pl.BlockSpec((bm, D), lambda i: (i, 0)),
        ],
        out_specs=pl.BlockSpec((bm, D), lambda i: (i, 0)),
        out_shape=jax.ShapeDtypeStruct((N, D), jnp.float32),
    )(a, b)


def _sc_agg_body(h_hbm, e_hbm, src_hbm, dst_hbm, out_hbm,
                 e_buf, h_buf, sidx, didx, acc,
                 sem_e, sem_h, sem_i, sem_s):
    c = lax.axis_index("c")
    s = lax.axis_index("s")
    w = s * 2 + c
    base = w * NCHW
    zcopies = [CHUNK] * (RPT // CHUNK) + ([RPT % CHUNK] if RPT % CHUNK else [])

    def zrow(r, _):
        for k in range(D // 16):
            e_buf[0, r, pl.ds(k * 16, 16)] = jnp.zeros((16,), jnp.float32)
        return 0
    lax.fori_loop(0, CHUNK, zrow, 0)
    off = 0
    for zc in zcopies:
        pltpu.sync_copy(e_buf.at[0].at[pl.ds(0, zc)],
                        acc.at[pl.ds(s * RPT + off, zc)])
        off += zc
    plsc.subcore_barrier()

    def issue_e(t):
        pltpu.async_copy(
            e_hbm.at[pl.ds(jnp.minimum(base + t, NCHUNKS - 1) * CHUNK, CHUNK)],
            e_buf.at[t % 2], sem_e.at[t % 2])

    def wait_e(t):
        pltpu.make_async_copy(e_hbm.at[pl.ds(0, CHUNK)], e_buf.at[t % 2],
                              sem_e.at[t % 2]).wait()

    def issue_idx(t):
        cid = jnp.minimum(base + t, NCHP - 1)
        j = t % 3
        pltpu.async_copy(src_hbm.at[cid], sidx.at[j], sem_i.at[j])
        pltpu.async_copy(dst_hbm.at[cid], didx.at[j], sem_i.at[j])

    def wait_idx(t):
        j = t % 3
        pltpu.make_async_copy(src_hbm.at[0], sidx.at[j], sem_i.at[j]).wait()
        pltpu.make_async_copy(dst_hbm.at[0], didx.at[j], sem_i.at[j]).wait()

    def issue_gather(t):
        wait_idx(t)
        pltpu.async_copy(h_hbm.at[sidx.at[t % 3]], h_buf, sem_h)

    def wait_gather(t):
        pltpu.make_async_copy(h_hbm.at[sidx.at[0]], h_buf, sem_h).wait()

    def wait_scatter(t):
        pltpu.make_async_copy(e_buf.at[t % 2], acc.at[didx.at[t % 3]],
                              sem_s.at[t % 2]).wait()

    issue_e(0)
    issue_idx(0)
    issue_idx(1)
    issue_gather(0)

    def compute(slot):
        def mrow(r, _):
            for k in range(D // 16):
                sl = pl.ds(k * 16, 16)
                e_buf[slot, r, sl] = jnp.maximum(
                    e_buf[slot, r, sl] + h_buf[r, sl], 0.0)
            return 0
        lax.fori_loop(0, CHUNK, mrow, 0)

    @pl.loop(0, NCHW, step=2)
    def phase(t0):
        for b in range(2):
            t = t0 + b
            wait_gather(t)
            wait_e(t)
            compute(b)
            issue_gather(t + 1)
            pltpu.async_copy(e_buf.at[b], acc.at[didx.at[t % 3]],
                             sem_s.at[b], add=True)
            if b == 0:
                @pl.when(t0 >= 1)
                def _():
                    wait_scatter(t0 - 1)
            else:
                wait_scatter(t0)
            issue_e(t + 1)
            issue_idx(t + 2)

    wait_e(NCHW)
    wait_gather(NCHW)
    wait_idx(NCHW + 1)
    wait_scatter(NCHW - 1)
    plsc.subcore_barrier()

    off = 0
    for zc in zcopies:
        rows = pl.ds(s * RPT + off, zc)
        pltpu.sync_copy(acc.at[rows], out_hbm.at[c].at[rows])
        off += zc


def _sc_agg(h, e, src2d, dst2d):
    mesh = plsc.VectorSubcoreMesh(core_axis_name="c", subcore_axis_name="s")
    f = pl.kernel(
        _sc_agg_body,
        mesh=mesh,
        out_type=jax.ShapeDtypeStruct((2, NACC, D), jnp.float32),
        scratch_types=[
            pltpu.VMEM((2, CHUNK, D), jnp.float32),
            pltpu.VMEM((CHUNK, D), jnp.float32),
            pltpu.VMEM((3, CHUNK), jnp.int32),
            pltpu.VMEM((3, CHUNK), jnp.int32),
            pltpu.VMEM_SHARED((NACC, D), jnp.float32),
            pltpu.SemaphoreType.DMA((2,)),
            pltpu.SemaphoreType.DMA,
            pltpu.SemaphoreType.DMA((3,)),
            pltpu.SemaphoreType.DMA((2,)),
        ],
    )
    return f(h, e, src2d, dst2d)


def kernel(x, edge_index, edge_attr, W_atom, W_bond, Wn, We, node_rand, edge_rand):
    pad = NCHP * CHUNK - E
    iot = jnp.arange(pad, dtype=jnp.int32)
    src2d = jnp.concatenate(
        [edge_index[0], iot % N]).reshape(NCHP, CHUNK)
    dst2d = jnp.concatenate(
        [edge_index[1], N + (iot % (NACC - N))]).reshape(NCHP, CHUNK)
    node_mask = (node_rand > P_NODE).astype(jnp.float32)[:, None]
    edge_mask = (edge_rand > P_EDGE).astype(jnp.float32)[:, None]

    h0 = _embed(x, W_atom, node_mask, bm=2000)
    e0 = _embed(edge_attr, W_bond, edge_mask, bm=4000)

    def mp(h, e, i):
        parts = _sc_agg(h, e, src2d, dst2d)
        return _h_update(parts, Wn[i], h), _e_update(e, We[i])

    h1, e1 = mp(h0, e0, 0)
    h2, e2 = mp(h1, e1, 1)
    g = _add2(h2, h1, 1.0)
    u00h, u00e = mp(g, e2, 3)
    xs0 = _add2(h1, u00h, 1.0)
    h3, e3 = mp(h2, e2, 2)
    g = _add2(h3, h2, 1.0)
    u11h, u11e = mp(g, e3, 5)
    g = _add2(u11h, xs0, 0.5)
    u10h, u10e = mp(g, u11e, 4)

    return (jnp.stack([u00h, u10h]), jnp.stack([u00e, u10e]),
            node_mask.reshape(-1), edge_mask.reshape(-1))


# speedup vs baseline: 1.2155x; 1.0708x over previous
import functools

import jax
import jax.numpy as jnp
from jax import lax
from jax.experimental import pallas as pl
from jax.experimental.pallas import tpu as pltpu
from jax.experimental.pallas import tpu_sc as plsc

N = 10000
E = 320000
DIN = 128
DE = 16
D = 128
P_NODE = 0.1
P_EDGE = 0.1

NACC = 10112
CHUNK = 128
NW = 32
NCHW = 80
NCHUNKS = E // CHUNK
NCHP = NW * NCHW
RPT = NACC // 16


def _embed_body(a_ref, w_ref, m_ref, o_ref):
    o_ref[...] = jnp.dot(a_ref[...], w_ref[...],
                         preferred_element_type=jnp.float32) * m_ref[...]


def _embed(a, w, m, bm):
    r, k = a.shape
    return pl.pallas_call(
        _embed_body,
        grid=(r // bm,),
        in_specs=[
            pl.BlockSpec((bm, k), lambda i: (i, 0)),
            pl.BlockSpec((k, D), lambda i: (0, 0)),
            pl.BlockSpec((bm, 1), lambda i: (i, 0)),
        ],
        out_specs=pl.BlockSpec((bm, D), lambda i: (i, 0)),
        out_shape=jax.ShapeDtypeStruct((r, D), jnp.float32),
    )(a, w, m)


def _eupd_body(e_ref, w_ref, o_ref):
    e = e_ref[...]
    o_ref[...] = jnp.maximum(
        jnp.dot(e, w_ref[...], preferred_element_type=jnp.float32), 0.0) + e


def _e_update(e, w, bm=4000):
    return pl.pallas_call(
        _eupd_body,
        grid=(E // bm,),
        in_specs=[
            pl.BlockSpec((bm, D), lambda i: (i, 0)),
            pl.BlockSpec((D, D), lambda i: (0, 0)),
        ],
        out_specs=pl.BlockSpec((bm, D), lambda i: (i, 0)),
        out_shape=jax.ShapeDtypeStruct((E, D), jnp.float32),
    )(e, w)


def _hupd_body(p_ref, w_ref, r_ref, o_ref):
    agg = p_ref[0] + p_ref[1]
    o_ref[...] = jnp.maximum(
        jnp.dot(agg, w_ref[...], preferred_element_type=jnp.float32),
        0.0) + r_ref[...]


def _h_update(parts, w, res, bm=2000):
    return pl.pallas_call(
        _hupd_body,
        grid=(N // bm,),
        in_specs=[
            pl.BlockSpec((2, bm, D), lambda i: (0, i, 0)),
            pl.BlockSpec((D, D), lambda i: (0, 0)),
            pl.BlockSpec((bm, D), lambda i: (i, 0)),
        ],
        out_specs=pl.BlockSpec((bm, D), lambda i: (i, 0)),
        out_shape=jax.ShapeDtypeStruct((N, D), jnp.float32),
    )(parts, w, res)


def _add2_body(a_ref, b_ref, o_ref, *, scale):
    o_ref[...] = a_ref[...] + b_ref[...] * scale


def _add2(a, b, scale, bm=2000):
    return pl.pallas_call(
        functools.partial(_add2_body, scale=scale),
        grid=(N // bm,),
        in_specs=[
            pl.BlockSpec((bm, D), lambda i: (i, 0)),
            pl.BlockSpec((bm, D), lambda i: (i, 0)),
        ],
        out_specs=pl.BlockSpec((bm, D), lambda i: (i, 0)),
        out_shape=jax.ShapeDtypeStruct((N, D), jnp.float32),
    )(a, b)


def _sc_agg_body(h_hbm, e_hbm, src_hbm, dst_hbm, out_hbm,
                 e_buf, h_buf, sidx, didx, acc,
                 sem_e, sem_h, sem_i, sem_s):
    c = lax.axis_index("c")
    s = lax.axis_index("s")
    w = s * 2 + c
    base = w * NCHW
    zcopies = [CHUNK] * (RPT // CHUNK) + ([RPT % CHUNK] if RPT % CHUNK else [])

    def zrow(r, _):
        for k in range(D // 16):
            e_buf[0, r, pl.ds(k * 16, 16)] = jnp.zeros((16,), jnp.float32)
        return 0
    lax.fori_loop(0, CHUNK, zrow, 0)
    off = 0
    for zc in zcopies:
        pltpu.sync_copy(e_buf.at[0].at[pl.ds(0, zc)],
                        acc.at[pl.ds(s * RPT + off, zc)])
        off += zc
    plsc.subcore_barrier()

    def issue_e(t):
        pltpu.async_copy(
            e_hbm.at[pl.ds(jnp.minimum(base + t, NCHUNKS - 1) * CHUNK, CHUNK)],
            e_buf.at[t % 2], sem_e.at[t % 2])

    def wait_e(t):
        pltpu.make_async_copy(e_hbm.at[pl.ds(0, CHUNK)], e_buf.at[t % 2],
                              sem_e.at[t % 2]).wait()

    def issue_idx(t):
        cid = jnp.minimum(base + t, NCHP - 1)
        j = t % 3
        pltpu.async_copy(src_hbm.at[cid], sidx.at[j], sem_i.at[j])
        pltpu.async_copy(dst_hbm.at[cid], didx.at[j], sem_i.at[j])

    def wait_idx(t):
        j = t % 3
        pltpu.make_async_copy(src_hbm.at[0], sidx.at[j], sem_i.at[j]).wait()
        pltpu.make_async_copy(dst_hbm.at[0], didx.at[j], sem_i.at[j]).wait()

    def issue_gather(t):
        wait_idx(t)
        pltpu.async_copy(h_hbm.at[sidx.at[t % 3]], h_buf, sem_h)

    def wait_gather(t):
        pltpu.make_async_copy(h_hbm.at[sidx.at[0]], h_buf, sem_h).wait()

    def wait_scatter(t):
        pltpu.make_async_copy(e_buf.at[t % 2], acc.at[didx.at[t % 3]],
                              sem_s.at[t % 2]).wait()

    issue_e(0)
    issue_idx(0)
    issue_idx(1)
    issue_gather(0)

    def compute(slot):
        def mrow(r, _):
            for k in range(D // 16):
                sl = pl.ds(k * 16, 16)
                e_buf[slot, r, sl] = jnp.maximum(
                    e_buf[slot, r, sl] + h_buf[r, sl], 0.0)
            return 0
        lax.fori_loop(0, CHUNK, mrow, 0)

    @pl.loop(0, NCHW, step=2)
    def phase(t0):
        for b in range(2):
            t = t0 + b
            wait_gather(t)
            wait_e(t)
            if b == 0:
                @pl.when(t0 >= 1)
                def _():
                    wait_scatter(t0 - 1)
            else:
                wait_scatter(t0)
            issue_e(t + 1)
            issue_idx(t + 2)
            compute(b)
            issue_gather(t + 1)
            pltpu.async_copy(e_buf.at[b], acc.at[didx.at[t % 3]],
                             sem_s.at[b], add=True)

    wait_e(NCHW)
    wait_gather(NCHW)
    wait_idx(NCHW + 1)
    wait_scatter(NCHW - 1)
    plsc.subcore_barrier()

    off = 0
    for zc in zcopies:
        rows = pl.ds(s * RPT + off, zc)
        pltpu.sync_copy(acc.at[rows], out_hbm.at[c].at[rows])
        off += zc


def _sc_agg(h, e, src2d, dst2d):
    mesh = plsc.VectorSubcoreMesh(core_axis_name="c", subcore_axis_name="s")
    f = pl.kernel(
        _sc_agg_body,
        mesh=mesh,
        out_type=jax.ShapeDtypeStruct((2, NACC, D), jnp.float32),
        scratch_types=[
            pltpu.VMEM((2, CHUNK, D), jnp.float32),
            pltpu.VMEM((CHUNK, D), jnp.float32),
            pltpu.VMEM((3, CHUNK), jnp.int32),
            pltpu.VMEM((3, CHUNK), jnp.int32),
            pltpu.VMEM_SHARED((NACC, D), jnp.float32),
            pltpu.SemaphoreType.DMA((2,)),
            pltpu.SemaphoreType.DMA,
            pltpu.SemaphoreType.DMA((3,)),
            pltpu.SemaphoreType.DMA((2,)),
        ],
    )
    return f(h, e, src2d, dst2d)


def kernel(x, edge_index, edge_attr, W_atom, W_bond, Wn, We, node_rand, edge_rand):
    pad = NCHP * CHUNK - E
    iot = jnp.arange(pad, dtype=jnp.int32)
    src2d = jnp.concatenate(
        [edge_index[0], iot % N]).reshape(NCHP, CHUNK)
    dst2d = jnp.concatenate(
        [edge_index[1], N + (iot % (NACC - N))]).reshape(NCHP, CHUNK)
    node_mask = (node_rand > P_NODE).astype(jnp.float32)[:, None]
    edge_mask = (edge_rand > P_EDGE).astype(jnp.float32)[:, None]

    h0 = _embed(x, W_atom, node_mask, bm=2000)
    e0 = _embed(edge_attr, W_bond, edge_mask, bm=4000)

    def mp(h, e, i):
        parts = _sc_agg(h, e, src2d, dst2d)
        return _h_update(parts, Wn[i], h), _e_update(e, We[i])

    h1, e1 = mp(h0, e0, 0)
    h2, e2 = mp(h1, e1, 1)
    g = _add2(h2, h1, 1.0)
    u00h, u00e = mp(g, e2, 3)
    xs0 = _add2(h1, u00h, 1.0)
    h3, e3 = mp(h2, e2, 2)
    g = _add2(h3, h2, 1.0)
    u11h, u11e = mp(g, e3, 5)
    g = _add2(u11h, xs0, 0.5)
    u10h, u10e = mp(g, u11e, 4)

    return (jnp.stack([u00h, u10h]), jnp.stack([u00e, u10e]),
            node_mask.reshape(-1), edge_mask.reshape(-1))


# speedup vs baseline: 1.2539x; 1.0316x over previous
import functools

import jax
import jax.numpy as jnp
from jax import lax
from jax.experimental import pallas as pl
from jax.experimental.pallas import tpu as pltpu
from jax.experimental.pallas import tpu_sc as plsc

N = 10000
E = 320000
DIN = 128
DE = 16
D = 128
P_NODE = 0.1
P_EDGE = 0.1

NACC = 10112
CHUNK = 128
NW = 32
NCHW = 80
NCHUNKS = E // CHUNK
NCHP = NW * NCHW
RPT = NACC // 16


def _embed_body(a_ref, w_ref, m_ref, o_ref):
    o_ref[...] = jnp.dot(a_ref[...], w_ref[...],
                         preferred_element_type=jnp.float32) * m_ref[...]


def _embed(a, w, m, bm):
    r, k = a.shape
    return pl.pallas_call(
        _embed_body,
        grid=(r // bm,),
        in_specs=[
            pl.BlockSpec((bm, k), lambda i: (i, 0)),
            pl.BlockSpec((k, D), lambda i: (0, 0)),
            pl.BlockSpec((bm, 1), lambda i: (i, 0)),
        ],
        out_specs=pl.BlockSpec((bm, D), lambda i: (i, 0)),
        out_shape=jax.ShapeDtypeStruct((r, D), jnp.float32),
    )(a, w, m)


def _eupd_body(e_ref, w_ref, o_ref):
    e = e_ref[...]
    o_ref[...] = jnp.maximum(
        jnp.dot(e, w_ref[...], preferred_element_type=jnp.float32), 0.0) + e


def _e_update(e, w, bm=4000):
    return pl.pallas_call(
        _eupd_body,
        grid=(E // bm,),
        in_specs=[
            pl.BlockSpec((bm, D), lambda i: (i, 0)),
            pl.BlockSpec((D, D), lambda i: (0, 0)),
        ],
        out_specs=pl.BlockSpec((bm, D), lambda i: (i, 0)),
        out_shape=jax.ShapeDtypeStruct((E, D), jnp.float32),
    )(e, w)


def _hupd_body(p_ref, w_ref, r_ref, o_ref):
    agg = p_ref[0] + p_ref[1]
    o_ref[...] = jnp.maximum(
        jnp.dot(agg, w_ref[...], preferred_element_type=jnp.float32),
        0.0) + r_ref[...]


def _h_update(parts, w, res, bm=2000):
    return pl.pallas_call(
        _hupd_body,
        grid=(N // bm,),
        in_specs=[
            pl.BlockSpec((2, bm, D), lambda i: (0, i, 0)),
            pl.BlockSpec((D, D), lambda i: (0, 0)),
            pl.BlockSpec((bm, D), lambda i: (i, 0)),
        ],
        out_specs=pl.BlockSpec((bm, D), lambda i: (i, 0)),
        out_shape=jax.ShapeDtypeStruct((N, D), jnp.float32),
    )(parts, w, res)


def _add2_body(a_ref, b_ref, o_ref, *, scale):
    o_ref[...] = a_ref[...] + b_ref[...] * scale


def _add2(a, b, scale, bm=2000):
    return pl.pallas_call(
        functools.partial(_add2_body, scale=scale),
        grid=(N // bm,),
        in_specs=[
            pl.BlockSpec((bm, D), lambda i: (i, 0)),
            pl.BlockSpec((bm, D), lambda i: (i, 0)),
        ],
        out_specs=pl.BlockSpec((bm, D), lambda i: (i, 0)),
        out_shape=jax.ShapeDtypeStruct((N, D), jnp.float32),
    )(a, b)


def _sc_agg_body(h_hbm, e_hbm, src_hbm, dst_hbm, out_hbm,
                 e_buf, h_buf, sidx, didx, acc,
                 sem_e, sem_h, sem_i, sem_s):
    c = lax.axis_index("c")
    s = lax.axis_index("s")
    w = s * 2 + c
    base = w * NCHW
    zcopies = [CHUNK] * (RPT // CHUNK) + ([RPT % CHUNK] if RPT % CHUNK else [])

    def zrow(r, _):
        for k in range(D // 16):
            e_buf[0, r, pl.ds(k * 16, 16)] = jnp.zeros((16,), jnp.float32)
        return 0
    lax.fori_loop(0, CHUNK, zrow, 0)
    off = 0
    for zc in zcopies:
        pltpu.sync_copy(e_buf.at[0].at[pl.ds(0, zc)],
                        acc.at[pl.ds(s * RPT + off, zc)])
        off += zc
    plsc.subcore_barrier()

    def issue_e(t):
        pltpu.async_copy(
            e_hbm.at[pl.ds(jnp.minimum(base + t, NCHUNKS - 1) * CHUNK, CHUNK)],
            e_buf.at[t % 2], sem_e.at[t % 2])

    def wait_e(t):
        pltpu.make_async_copy(e_hbm.at[pl.ds(0, CHUNK)], e_buf.at[t % 2],
                              sem_e.at[t % 2]).wait()

    HC = CHUNK // 2

    def issue_idx(t):
        cid = jnp.minimum(base + t, NCHP - 1)
        j = t % 3
        pltpu.async_copy(src_hbm.at[cid], sidx.at[j], sem_i.at[j])
        pltpu.async_copy(dst_hbm.at[cid], didx.at[j], sem_i.at[j])

    def wait_idx(t):
        j = t % 3
        pltpu.make_async_copy(src_hbm.at[0], sidx.at[j], sem_i.at[j]).wait()
        pltpu.make_async_copy(dst_hbm.at[0], didx.at[j], sem_i.at[j]).wait()

    def issue_gather_half(t, half):
        pltpu.async_copy(h_hbm.at[sidx.at[t % 3].at[pl.ds(half * HC, HC)]],
                         h_buf.at[pl.ds(half * HC, HC)], sem_h)

    def wait_gather(t):
        for half in range(2):
            pltpu.make_async_copy(h_hbm.at[sidx.at[0].at[pl.ds(0, HC)]],
                                  h_buf.at[pl.ds(half * HC, HC)],
                                  sem_h).wait()

    def issue_scatter(t, slot):
        pltpu.async_copy(e_buf.at[slot], acc.at[didx.at[t % 3]],
                         sem_s.at[slot], add=True)

    def wait_scatter(t):
        pltpu.make_async_copy(e_buf.at[t % 2], acc.at[didx.at[0]],
                              sem_s.at[t % 2]).wait()

    def compute_half(slot, half):
        def mrow(r, _):
            for k in range(D // 16):
                sl = pl.ds(k * 16, 16)
                e_buf[slot, r, sl] = jnp.maximum(
                    e_buf[slot, r, sl] + h_buf[r, sl], 0.0)
            return 0
        lax.fori_loop(half * HC, (half + 1) * HC, mrow, 0)

    issue_e(0)
    issue_idx(0)
    issue_idx(1)
    wait_idx(0)
    issue_gather_half(0, 0)
    issue_gather_half(0, 1)

    @pl.loop(0, NCHW, step=2)
    def phase(t0):
        for b in range(2):
            t = t0 + b
            wait_gather(t)
            wait_e(t)
            if b == 0:
                @pl.when(t0 >= 1)
                def _():
                    wait_scatter(t0 - 1)
            else:
                wait_scatter(t0)
            issue_e(t + 1)
            issue_idx(t + 2)
            wait_idx(t + 1)
            compute_half(b, 0)
            issue_gather_half(t + 1, 0)
            compute_half(b, 1)
            issue_gather_half(t + 1, 1)
            issue_scatter(t, b)

    wait_e(NCHW)
    wait_gather(NCHW)
    wait_idx(NCHW + 1)
    wait_scatter(NCHW - 1)
    plsc.subcore_barrier()

    off = 0
    for zc in zcopies:
        rows = pl.ds(s * RPT + off, zc)
        pltpu.sync_copy(acc.at[rows], out_hbm.at[c].at[rows])
        off += zc


def _sc_agg(h, e, src2d, dst2d):
    mesh = plsc.VectorSubcoreMesh(core_axis_name="c", subcore_axis_name="s")
    f = pl.kernel(
        _sc_agg_body,
        mesh=mesh,
        out_type=jax.ShapeDtypeStruct((2, NACC, D), jnp.float32),
        scratch_types=[
            pltpu.VMEM((2, CHUNK, D), jnp.float32),
            pltpu.VMEM((CHUNK, D), jnp.float32),
            pltpu.VMEM((3, CHUNK), jnp.int32),
            pltpu.VMEM((3, CHUNK), jnp.int32),
            pltpu.VMEM_SHARED((NACC, D), jnp.float32),
            pltpu.SemaphoreType.DMA((2,)),
            pltpu.SemaphoreType.DMA,
            pltpu.SemaphoreType.DMA((3,)),
            pltpu.SemaphoreType.DMA((2,)),
        ],
    )
    return f(h, e, src2d, dst2d)


def kernel(x, edge_index, edge_attr, W_atom, W_bond, Wn, We, node_rand, edge_rand):
    pad = NCHP * CHUNK - E
    iot = jnp.arange(pad, dtype=jnp.int32)
    src2d = jnp.concatenate(
        [edge_index[0], iot % N]).reshape(NCHP, CHUNK)
    dst2d = jnp.concatenate(
        [edge_index[1], N + (iot % (NACC - N))]).reshape(NCHP, CHUNK)
    node_mask = (node_rand > P_NODE).astype(jnp.float32)[:, None]
    edge_mask = (edge_rand > P_EDGE).astype(jnp.float32)[:, None]

    h0 = _embed(x, W_atom, node_mask, bm=2000)
    e0 = _embed(edge_attr, W_bond, edge_mask, bm=4000)

    def mp(h, e, i):
        parts = _sc_agg(h, e, src2d, dst2d)
        return _h_update(parts, Wn[i], h), _e_update(e, We[i])

    h1, e1 = mp(h0, e0, 0)
    h2, e2 = mp(h1, e1, 1)
    g = _add2(h2, h1, 1.0)
    u00h, u00e = mp(g, e2, 3)
    xs0 = _add2(h1, u00h, 1.0)
    h3, e3 = mp(h2, e2, 2)
    g = _add2(h3, h2, 1.0)
    u11h, u11e = mp(g, e3, 5)
    g = _add2(u11h, xs0, 0.5)
    u10h, u10e = mp(g, u11e, 4)

    return (jnp.stack([u00h, u10h]), jnp.stack([u00e, u10e]),
            node_mask.reshape(-1), edge_mask.reshape(-1))
